# Initial kernel scaffold; baseline (speedup 1.0000x reference)
import jax
import jax.numpy as jnp
from jax.experimental import pallas as pl


def kernel(x, edge_index, edge_attr, Wn1, bn1, Wn2, bn2, We1, be1, We2, be2, We3, be3, Wc1, bc1, Wc2, bc2, Wfi1, bfi1, Wfi2, bfi2, Wfo1, bfo1, Wfo2, bfo2, Wnm, bnm, Wcl1, bcl1, Wcl2, bcl2):
    raise NotImplementedError("write your pallas kernel here")


# speedup vs baseline: 2.2179x; 2.2179x over previous
import functools

import jax
import jax.numpy as jnp
from jax import lax
from jax.experimental import pallas as pl
from jax.experimental.pallas import tpu as pltpu
from jax.experimental.pallas import tpu_sc as plsc

N_NODES_C = 10000
N_EDGES_C = 320000

_NC = 2
_NS = 16
_NW = _NC * _NS
_EW = N_EDGES_C // _NW
_G = 80
_GI = _EW // _G
_GRP = 5
_CH = _G * _GRP
_KO = _GI // _GRP
_NN_PAD = 10240
_SEG = _NN_PAD // _NS

@functools.lru_cache(maxsize=None)
def _sc_kernels():
    mesh = plsc.VectorSubcoreMesh(core_axis_name="c", subcore_axis_name="s")
    sc_params = pltpu.CompilerParams(use_tc_tiling_on_sc=False)

    @functools.partial(
        pl.kernel,
        mesh=mesh,
        out_type=[
            jax.ShapeDtypeStruct((N_EDGES_C, 32), jnp.float32),
            jax.ShapeDtypeStruct((N_EDGES_C, 32), jnp.float32),
        ],
        scratch_types=[
            pltpu.VMEM((_GI, _G), jnp.int32),
            pltpu.VMEM((_GI, _G), jnp.int32),
            pltpu.VMEM((_CH, 32), jnp.float32),
            pltpu.VMEM((_CH, 32), jnp.float32),
            pltpu.SemaphoreType.DMA,
            pltpu.SemaphoreType.DMA,
        ],
        compiler_params=sc_params,
    )
    def _sc_gather(node_hbm, row2_hbm, col2_hbm, nrow_hbm, ncol_hbm,
                   idxr, idxc, bufr, bufc, semr, semc):
        cid = lax.axis_index("c")
        sid = lax.axis_index("s")
        wid = sid * _NC + cid
        pltpu.sync_copy(row2_hbm.at[wid], idxr)
        pltpu.sync_copy(col2_hbm.at[wid], idxc)

        def outer(k, carry):
            base_e = wid * _EW + k * _CH
            waits = []
            for g in range(_GRP):
                j = k * _GRP + g
                waits.append(pltpu.async_copy(
                    node_hbm.at[idxr.at[j]], bufr.at[pl.ds(g * _G, _G)], semr))
                waits.append(pltpu.async_copy(
                    node_hbm.at[idxc.at[j]], bufc.at[pl.ds(g * _G, _G)], semc))
            for w in waits:
                w.wait()
            pltpu.sync_copy(bufr, nrow_hbm.at[pl.ds(base_e, _CH)])
            pltpu.sync_copy(bufc, ncol_hbm.at[pl.ds(base_e, _CH)])
            return carry

        lax.fori_loop(0, _KO, outer, 0)

    @functools.partial(
        pl.kernel,
        mesh=mesh,
        out_type=[
            jax.ShapeDtypeStruct((_NC, _NN_PAD, 32), jnp.float32),
            jax.ShapeDtypeStruct((_NC, _NN_PAD, 32), jnp.float32),
        ],
        scratch_types=[
            pltpu.VMEM((_GI, _G), jnp.int32),
            pltpu.VMEM((_CH, 32), jnp.float32),
            pltpu.VMEM((_CH, 32), jnp.float32),
            pltpu.VMEM((_SEG, 32), jnp.float32),
            pltpu.VMEM_SHARED((_NN_PAD, 32), jnp.float32),
            pltpu.VMEM_SHARED((_NN_PAD, 32), jnp.float32),
        ],
        compiler_params=sc_params,
    )
    def _sc_scatter(fo_hbm, fi_hbm, row2_hbm, zeros_hbm, outfo_hbm, outfi_hbm,
                    idxr, vfo, vfi, zbuf, acc_fo, acc_fi):
        cid = lax.axis_index("c")
        sid = lax.axis_index("s")
        wid = sid * _NC + cid
        pltpu.sync_copy(zeros_hbm, zbuf)
        pltpu.sync_copy(zbuf, acc_fo.at[pl.ds(sid * _SEG, _SEG)])
        pltpu.sync_copy(zbuf, acc_fi.at[pl.ds(sid * _SEG, _SEG)])
        pltpu.sync_copy(row2_hbm.at[wid], idxr)
        plsc.subcore_barrier()

        def outer(k, carry):
            base_e = wid * _EW + k * _CH
            pltpu.sync_copy(fo_hbm.at[pl.ds(base_e, _CH)], vfo)
            pltpu.sync_copy(fi_hbm.at[pl.ds(base_e, _CH)], vfi)
            for g in range(_GRP):
                j = k * _GRP + g
                pltpu.sync_copy(vfo.at[pl.ds(g * _G, _G)],
                                acc_fo.at[idxr.at[j]], add=True)
                pltpu.sync_copy(vfi.at[pl.ds(g * _G, _G)],
                                acc_fi.at[idxr.at[j]], add=True)
            return carry

        lax.fori_loop(0, _KO, outer, 0)
        plsc.subcore_barrier()
        pltpu.sync_copy(acc_fo.at[pl.ds(sid * _SEG, _SEG)], zbuf)
        pltpu.sync_copy(zbuf, outfo_hbm.at[cid, pl.ds(sid * _SEG, _SEG)])
        pltpu.sync_copy(acc_fi.at[pl.ds(sid * _SEG, _SEG)], zbuf)
        pltpu.sync_copy(zbuf, outfi_hbm.at[cid, pl.ds(sid * _SEG, _SEG)])

    return _sc_gather, _sc_scatter


_BE = 6400
_GRID = N_EDGES_C // _BE


def _tc_node_enc(x_ref, w1, b1, w2, b2, o_ref):
    h = jnp.maximum(jnp.dot(x_ref[...], w1[...],
                            preferred_element_type=jnp.float32) + b1[...], 0.0)
    o_ref[...] = jnp.maximum(jnp.dot(h, w2[...],
                                     preferred_element_type=jnp.float32) + b2[...], 0.0)


def _tc_step1(ea, nrow, ncol, rv, cv,
              we1, be1, we2, be2, we3, be3,
              a1, a2, a34, bc1, wc2, bc2,
              p1, q1, bt, bd, bt2,
              e1_ref, fo_ref, fi_ref):
    f32 = jnp.float32
    ie = jnp.maximum(jnp.dot(ea[...], we1[...], preferred_element_type=f32) + be1[...], 0.0)
    ie = jnp.maximum(jnp.dot(ie, we2[...], preferred_element_type=f32) + be2[...], 0.0)
    ie = jnp.maximum(jnp.dot(ie, we3[...], preferred_element_type=f32) + be3[...], 0.0)
    h = (jnp.dot(nrow[...], a1[...], preferred_element_type=f32)
         + jnp.dot(ncol[...], a2[...], preferred_element_type=f32)
         + jnp.dot(ie, a34[...], preferred_element_type=f32) + bc1[...])
    h = jnp.maximum(h, 0.0)
    e1 = jnp.maximum(jnp.dot(h, wc2[...], preferred_element_type=f32) + bc2[...], 0.0)
    e1_ref[...] = e1
    tu = (jnp.dot(ncol[...], p1[...], preferred_element_type=f32)
          + jnp.dot(e1, q1[...], preferred_element_type=f32) + bt[...])
    tu = jnp.maximum(tu, 0.0)
    ff = jnp.maximum(jnp.dot(tu, bd[...], preferred_element_type=f32) + bt2[...], 0.0)
    r = rv[...]
    c = cv[...]
    mo = (r < c).astype(f32)
    mi = (r > c).astype(f32)
    fo_ref[...] = ff[:, :32] * mo
    fi_ref[...] = ff[:, 32:] * mi


def _tc_node_mlp(fop, fip, win, wout, bnm, o_ref):
    f32 = jnp.float32
    flow_in = (fip[0] + fip[1])[:N_NODES_C]
    flow_out = (fop[0] + fop[1])[:N_NODES_C]
    o_ref[...] = jnp.maximum(
        jnp.dot(flow_in, win[...], preferred_element_type=f32)
        + jnp.dot(flow_out, wout[...], preferred_element_type=f32) + bnm[...], 0.0)


def _tc_step2(ea, nrow, ncol, e1,
              we1, be1, we2, be2, we3, be3,
              a1, a2, a3, a4, bc1, wc2, bc2,
              wcl1, bcl1, wcl2, bcl2,
              out_ref):
    f32 = jnp.float32
    ie = jnp.maximum(jnp.dot(ea[...], we1[...], preferred_element_type=f32) + be1[...], 0.0)
    ie = jnp.maximum(jnp.dot(ie, we2[...], preferred_element_type=f32) + be2[...], 0.0)
    ie = jnp.maximum(jnp.dot(ie, we3[...], preferred_element_type=f32) + be3[...], 0.0)
    h = (jnp.dot(nrow[...], a1[...], preferred_element_type=f32)
         + jnp.dot(ncol[...], a2[...], preferred_element_type=f32)
         + jnp.dot(ie, a3[...], preferred_element_type=f32)
         + jnp.dot(e1[...], a4[...], preferred_element_type=f32) + bc1[...])
    h = jnp.maximum(h, 0.0)
    e2 = jnp.maximum(jnp.dot(h, wc2[...], preferred_element_type=f32) + bc2[...], 0.0)
    cc = jnp.maximum(jnp.dot(e2, wcl1[...], preferred_element_type=f32) + bcl1[...], 0.0)
    out_ref[...] = jnp.dot(cc, wcl2[...], preferred_element_type=f32) + bcl2[...]


def _full_spec(shape):
    return pl.BlockSpec(shape, lambda i: tuple(0 for _ in shape))


def _edge_spec(cols):
    return pl.BlockSpec((_BE, cols), lambda i: (i, 0))


def kernel(x, edge_index, edge_attr, Wn1, bn1, Wn2, bn2, We1, be1, We2, be2,
           We3, be3, Wc1, bc1, Wc2, bc2, Wfi1, bfi1, Wfi2, bfi2, Wfo1, bfo1,
           Wfo2, bfo2, Wnm, bnm, Wcl1, bcl1, Wcl2, bcl2):
    f32 = jnp.float32
    row = edge_index[0].astype(jnp.int32)
    col = edge_index[1].astype(jnp.int32)
    row2 = row.reshape(_NW, _GI, _G)
    col2 = col.reshape(_NW, _GI, _G)
    rv = row.reshape(N_EDGES_C, 1)
    cv = col.reshape(N_EDGES_C, 1)

    a1 = Wc1[0:32]
    a2 = Wc1[32:64]
    a3 = Wc1[64:80]
    a4 = Wc1[80:96]
    a34 = a3 + a4
    p1 = jnp.concatenate([Wfo1[:32], Wfi1[:32]], axis=1)
    q1 = jnp.concatenate([Wfo1[32:], Wfi1[32:]], axis=1)
    bt = jnp.concatenate([bfo1, bfi1])
    bd = jnp.zeros((112, 64), f32)
    bd = bd.at[:56, :32].set(Wfo2).at[56:, 32:].set(Wfi2)
    bt2 = jnp.concatenate([bfo2, bfi2])
    win = Wnm[:32]
    wout = Wnm[32:]
    zeros_seg = jnp.zeros((_SEG, 32), f32)

    node0 = pl.pallas_call(
        _tc_node_enc,
        out_shape=jax.ShapeDtypeStruct((N_NODES_C, 32), f32),
    )(x, Wn1, bn1, Wn2, bn2)

    sc_gather, sc_scatter = _sc_kernels()

    nrow0, ncol0 = sc_gather(node0, row2, col2)

    e1, fo, fi = pl.pallas_call(
        _tc_step1,
        grid=(_GRID,),
        in_specs=[
            _edge_spec(6), _edge_spec(32), _edge_spec(32),
            _edge_spec(1), _edge_spec(1),
            _full_spec(We1.shape), _full_spec(be1.shape),
            _full_spec(We2.shape), _full_spec(be2.shape),
            _full_spec(We3.shape), _full_spec(be3.shape),
            _full_spec(a1.shape), _full_spec(a2.shape), _full_spec(a34.shape),
            _full_spec(bc1.shape), _full_spec(Wc2.shape), _full_spec(bc2.shape),
            _full_spec(p1.shape), _full_spec(q1.shape), _full_spec(bt.shape),
            _full_spec(bd.shape), _full_spec(bt2.shape),
        ],
        out_specs=[_edge_spec(16), _edge_spec(32), _edge_spec(32)],
        out_shape=[
            jax.ShapeDtypeStruct((N_EDGES_C, 16), f32),
            jax.ShapeDtypeStruct((N_EDGES_C, 32), f32),
            jax.ShapeDtypeStruct((N_EDGES_C, 32), f32),
        ],
    )(edge_attr, nrow0, ncol0, rv, cv,
      We1, be1, We2, be2, We3, be3,
      a1, a2, a34, bc1, Wc2, bc2, p1, q1, bt, bd, bt2)

    fo_p, fi_p = sc_scatter(fo, fi, row2, zeros_seg)

    node1 = pl.pallas_call(
        _tc_node_mlp,
        out_shape=jax.ShapeDtypeStruct((N_NODES_C, 32), f32),
    )(fo_p, fi_p, win, wout, bnm)

    nrow1, ncol1 = sc_gather(node1, row2, col2)

    out = pl.pallas_call(
        _tc_step2,
        grid=(_GRID,),
        in_specs=[
            _edge_spec(6), _edge_spec(32), _edge_spec(32), _edge_spec(16),
            _full_spec(We1.shape), _full_spec(be1.shape),
            _full_spec(We2.shape), _full_spec(be2.shape),
            _full_spec(We3.shape), _full_spec(be3.shape),
            _full_spec(a1.shape), _full_spec(a2.shape),
            _full_spec(a3.shape), _full_spec(a4.shape),
            _full_spec(bc1.shape), _full_spec(Wc2.shape), _full_spec(bc2.shape),
            _full_spec(Wcl1.shape), _full_spec(bcl1.shape),
            _full_spec(Wcl2.shape), _full_spec(bcl2.shape),
        ],
        out_specs=[_edge_spec(1)],
        out_shape=[jax.ShapeDtypeStruct((N_EDGES_C, 1), f32)],
    )(edge_attr, nrow1, ncol1, e1,
      We1, be1, We2, be2, We3, be3,
      a1, a2, a3, a4, bc1, Wc2, bc2, Wcl1, bcl1, Wcl2, bcl2)[0]

    return out


# speedup vs baseline: 5.9173x; 2.6680x over previous
import functools

import jax
import jax.numpy as jnp
from jax import lax
from jax.experimental import pallas as pl
from jax.experimental.pallas import tpu as pltpu
from jax.experimental.pallas import tpu_sc as plsc

N_NODES_C = 10000
N_EDGES_C = 320000

_NC = 2
_NS = 16
_NW = _NC * _NS
_EW = N_EDGES_C // _NW
_G = 80
_GI = _EW // _G
_GRP = 5
_CH = _G * _GRP
_KO = _GI // _GRP
_NN_PAD = 10240
_SEG = _NN_PAD // _NS
_DUMP = 10208


@functools.lru_cache(maxsize=None)
def _sc_kernels():
    mesh = plsc.VectorSubcoreMesh(core_axis_name="c", subcore_axis_name="s")
    sc_params = pltpu.CompilerParams(use_tc_tiling_on_sc=False)

    @functools.partial(
        pl.kernel,
        mesh=mesh,
        out_type=[
            jax.ShapeDtypeStruct((N_EDGES_C, 32), jnp.float32),
            jax.ShapeDtypeStruct((N_EDGES_C, 32), jnp.float32),
        ],
        scratch_types=[
            pltpu.VMEM((_GI, _G), jnp.int32),
            pltpu.VMEM((_GI, _G), jnp.int32),
            pltpu.VMEM((_CH, 32), jnp.float32),
            pltpu.VMEM((_CH, 32), jnp.float32),
            pltpu.SemaphoreType.DMA,
            pltpu.SemaphoreType.DMA,
        ],
        compiler_params=sc_params,
    )
    def _sc_gather(node_hbm, row2_hbm, col2_hbm, nrow_hbm, ncol_hbm,
                   idxr, idxc, bufr, bufc, semr, semc):
        cid = lax.axis_index("c")
        sid = lax.axis_index("s")
        wid = sid * _NC + cid
        pltpu.sync_copy(row2_hbm.at[wid], idxr)
        pltpu.sync_copy(col2_hbm.at[wid], idxc)

        def outer(k, carry):
            base_e = wid * _EW + k * _CH
            waits = []
            for g in range(_GRP):
                j = k * _GRP + g
                waits.append(pltpu.async_copy(
                    node_hbm.at[idxr.at[j]], bufr.at[pl.ds(g * _G, _G)], semr))
                waits.append(pltpu.async_copy(
                    node_hbm.at[idxc.at[j]], bufc.at[pl.ds(g * _G, _G)], semc))
            for w in waits:
                w.wait()
            pltpu.sync_copy(bufr, nrow_hbm.at[pl.ds(base_e, _CH)])
            pltpu.sync_copy(bufc, ncol_hbm.at[pl.ds(base_e, _CH)])
            return carry

        lax.fori_loop(0, _KO, outer, 0)

    @functools.partial(
        pl.kernel,
        mesh=mesh,
        out_type=[
            jax.ShapeDtypeStruct((_NC, _NN_PAD, 32), jnp.float32),
            jax.ShapeDtypeStruct((_NC, _NN_PAD, 32), jnp.float32),
        ],
        scratch_types=[
            pltpu.VMEM((_GI, _G), jnp.int32),
            pltpu.VMEM((_GI, _G), jnp.int32),
            pltpu.VMEM((_CH, 32), jnp.float32),
            pltpu.VMEM((_CH, 32), jnp.float32),
            pltpu.VMEM((_SEG, 32), jnp.float32),
            pltpu.VMEM_SHARED((_NN_PAD, 32), jnp.float32),
            pltpu.VMEM_SHARED((_NN_PAD, 32), jnp.float32),
        ],
        compiler_params=sc_params,
    )
    def _sc_scatter(fo_hbm, fi_hbm, rfo_hbm, rfi_hbm, zeros_hbm,
                    outfo_hbm, outfi_hbm,
                    idxo, idxi, vfo, vfi, zbuf, acc_fo, acc_fi):
        cid = lax.axis_index("c")
        sid = lax.axis_index("s")
        wid = sid * _NC + cid
        pltpu.sync_copy(zeros_hbm, zbuf)
        pltpu.sync_copy(zbuf, acc_fo.at[pl.ds(sid * _SEG, _SEG)])
        pltpu.sync_copy(zbuf, acc_fi.at[pl.ds(sid * _SEG, _SEG)])
        pltpu.sync_copy(rfo_hbm.at[wid], idxo)
        pltpu.sync_copy(rfi_hbm.at[wid], idxi)
        plsc.subcore_barrier()

        def outer(k, carry):
            base_e = wid * _EW + k * _CH
            pltpu.sync_copy(fo_hbm.at[pl.ds(base_e, _CH)], vfo)
            pltpu.sync_copy(fi_hbm.at[pl.ds(base_e, _CH)], vfi)
            for g in range(_GRP):
                j = k * _GRP + g
                pltpu.sync_copy(vfo.at[pl.ds(g * _G, _G)],
                                acc_fo.at[idxo.at[j]], add=True)
                pltpu.sync_copy(vfi.at[pl.ds(g * _G, _G)],
                                acc_fi.at[idxi.at[j]], add=True)
            return carry

        lax.fori_loop(0, _KO, outer, 0)
        plsc.subcore_barrier()
        pltpu.sync_copy(acc_fo.at[pl.ds(sid * _SEG, _SEG)], zbuf)
        pltpu.sync_copy(zbuf, outfo_hbm.at[cid, pl.ds(sid * _SEG, _SEG)])
        pltpu.sync_copy(acc_fi.at[pl.ds(sid * _SEG, _SEG)], zbuf)
        pltpu.sync_copy(zbuf, outfi_hbm.at[cid, pl.ds(sid * _SEG, _SEG)])

    return _sc_gather, _sc_scatter


_BE = 6400
_BR = _BE // 4
_GRID = N_EDGES_C // _BE
_EP = N_EDGES_C // 4


def _tc_node_enc(x_ref, w1, b1, w2, b2, o_ref):
    h = jnp.maximum(jnp.dot(x_ref[...], w1[...],
                            preferred_element_type=jnp.float32) + b1[...], 0.0)
    o_ref[...] = jnp.maximum(jnp.dot(h, w2[...],
                                     preferred_element_type=jnp.float32) + b2[...], 0.0)


def _tc_step1(ea, nrow, ncol,
              wenc1, benc1, wenc2, benc2, wenc3, benc3,
              wmm1, bmm1, wmm2, bmm2, wmm3, bmm3, wmm4, bmm4,
              e1_ref, fo_ref, fi_ref):
    f32 = jnp.float32
    ie = jnp.maximum(jnp.dot(ea[...], wenc1[...], preferred_element_type=f32) + benc1[...], 0.0)
    ie = jnp.maximum(jnp.dot(ie, wenc2[...], preferred_element_type=f32) + benc2[...], 0.0)
    ie = jnp.maximum(jnp.dot(ie, wenc3[...], preferred_element_type=f32) + benc3[...], 0.0)
    x1 = jnp.concatenate([nrow[...], ncol[...], ie], axis=1)
    h = jnp.maximum(jnp.dot(x1, wmm1[...], preferred_element_type=f32) + bmm1[...], 0.0)
    e1 = jnp.maximum(jnp.dot(h, wmm2[...], preferred_element_type=f32) + bmm2[...], 0.0)
    e1_ref[...] = e1
    x3 = jnp.concatenate([ncol[...], e1], axis=1)
    tu = jnp.maximum(jnp.dot(x3, wmm3[...], preferred_element_type=f32) + bmm3[...], 0.0)
    ff = jnp.maximum(jnp.dot(tu, wmm4[...], preferred_element_type=f32) + bmm4[...], 0.0)
    fo_ref[...] = ff[:, :128]
    fi_ref[...] = ff[:, 128:]


def _tc_node_mlp(fop, fip, win4, wout4, bnm4, o_ref):
    f32 = jnp.float32
    flow_in = fip[0] + fip[1]
    flow_out = fop[0] + fop[1]
    o_ref[...] = jnp.maximum(
        jnp.dot(flow_in, win4[...], preferred_element_type=f32)
        + jnp.dot(flow_out, wout4[...], preferred_element_type=f32) + bnm4[...], 0.0)


def _tc_step2(ea, nrow, ncol, e1,
              wenc1, benc1, wenc2, benc2, wenc3, benc3,
              wmm1, bmm1, wmm2, bmm2, wcl1, bcl1, wcl2, bcl2,
              out_ref):
    f32 = jnp.float32
    ie = jnp.maximum(jnp.dot(ea[...], wenc1[...], preferred_element_type=f32) + benc1[...], 0.0)
    ie = jnp.maximum(jnp.dot(ie, wenc2[...], preferred_element_type=f32) + benc2[...], 0.0)
    ie = jnp.maximum(jnp.dot(ie, wenc3[...], preferred_element_type=f32) + benc3[...], 0.0)
    x1 = jnp.concatenate([nrow[...], ncol[...], ie, e1[...]], axis=1)
    h = jnp.maximum(jnp.dot(x1, wmm1[...], preferred_element_type=f32) + bmm1[...], 0.0)
    e2 = jnp.maximum(jnp.dot(h, wmm2[...], preferred_element_type=f32) + bmm2[...], 0.0)
    cc = jnp.maximum(jnp.dot(e2, wcl1[...], preferred_element_type=f32) + bcl1[...], 0.0)
    out_ref[...] = jnp.dot(cc, wcl2[...], preferred_element_type=f32) + bcl2[...]


def _full_spec(shape):
    return pl.BlockSpec(shape, lambda i: tuple(0 for _ in shape))


def _row_spec(cols):
    return pl.BlockSpec((_BR, cols), lambda i: (i, 0))


def _bd4(w):
    a, b = w.shape
    z = jnp.zeros((4 * a, 4 * b), jnp.float32)
    for g in range(4):
        z = z.at[g * a:(g + 1) * a, g * b:(g + 1) * b].set(w)
    return z


def kernel(x, edge_index, edge_attr, Wn1, bn1, Wn2, bn2, We1, be1, We2, be2,
           We3, be3, Wc1, bc1, Wc2, bc2, Wfi1, bfi1, Wfi2, bfi2, Wfo1, bfo1,
           Wfo2, bfo2, Wnm, bnm, Wcl1, bcl1, Wcl2, bcl2):
    f32 = jnp.float32
    row = edge_index[0].astype(jnp.int32)
    col = edge_index[1].astype(jnp.int32)
    row2 = row.reshape(_NW, _GI, _G)
    col2 = col.reshape(_NW, _GI, _G)
    dump = _DUMP + (jnp.arange(N_EDGES_C, dtype=jnp.int32) % 32)
    rfo = jnp.where(row < col, row, dump).reshape(_NW, _GI, _G)
    rfi = jnp.where(row > col, row, dump).reshape(_NW, _GI, _G)

    a1 = Wc1[0:32]
    a2 = Wc1[32:64]
    a3 = Wc1[64:80]
    a4 = Wc1[80:96]
    wenc1, benc1 = _bd4(We1), jnp.tile(be1, 4)
    wenc2, benc2 = _bd4(We2), jnp.tile(be2, 4)
    wenc3, benc3 = _bd4(We3), jnp.tile(be3, 4)
    wmm1 = jnp.concatenate([_bd4(a1), _bd4(a2), _bd4(a3 + a4)], axis=0)
    bmm1 = jnp.tile(bc1, 4)
    wmm1h = jnp.concatenate([_bd4(a1), _bd4(a2), _bd4(a3), _bd4(a4)], axis=0)
    wmm2, bmm2 = _bd4(Wc2), jnp.tile(bc2, 4)
    top = jnp.concatenate([Wfo1[:32], Wfi1[:32]], axis=1)
    bot = jnp.concatenate([Wfo1[32:], Wfi1[32:]], axis=1)
    wmm3 = jnp.concatenate([_bd4(top), _bd4(bot)], axis=0)
    bmm3 = jnp.tile(jnp.concatenate([bfo1, bfi1]), 4)
    wmm4 = jnp.zeros((448, 256), f32)
    for g in range(4):
        wmm4 = wmm4.at[112 * g:112 * g + 56, 32 * g:32 * g + 32].set(Wfo2)
        wmm4 = wmm4.at[112 * g + 56:112 * g + 112,
                       128 + 32 * g:128 + 32 * g + 32].set(Wfi2)
    bmm4 = jnp.concatenate([jnp.tile(bfo2, 4), jnp.tile(bfi2, 4)])
    win4, wout4 = _bd4(Wnm[:32]), _bd4(Wnm[32:])
    bnm4 = jnp.tile(bnm, 4)
    wcl1, bcl1_4 = _bd4(Wcl1), jnp.tile(bcl1, 4)
    wcl2, bcl2_4 = _bd4(Wcl2), jnp.tile(bcl2, 4)
    zeros_seg = jnp.zeros((_SEG, 32), f32)
    ea4 = edge_attr.reshape(_EP, 24)

    node0 = pl.pallas_call(
        _tc_node_enc,
        out_shape=jax.ShapeDtypeStruct((N_NODES_C, 32), f32),
    )(x, Wn1, bn1, Wn2, bn2)

    sc_gather, sc_scatter = _sc_kernels()

    nrow0, ncol0 = sc_gather(node0, row2, col2)
    nrow0 = nrow0.reshape(_EP, 128)
    ncol0 = ncol0.reshape(_EP, 128)

    e1, fo, fi = pl.pallas_call(
        _tc_step1,
        grid=(_GRID,),
        in_specs=[
            _row_spec(24), _row_spec(128), _row_spec(128),
            _full_spec(wenc1.shape), _full_spec(benc1.shape),
            _full_spec(wenc2.shape), _full_spec(benc2.shape),
            _full_spec(wenc3.shape), _full_spec(benc3.shape),
            _full_spec(wmm1.shape), _full_spec(bmm1.shape),
            _full_spec(wmm2.shape), _full_spec(bmm2.shape),
            _full_spec(wmm3.shape), _full_spec(bmm3.shape),
            _full_spec(wmm4.shape), _full_spec(bmm4.shape),
        ],
        out_specs=[_row_spec(64), _row_spec(128), _row_spec(128)],
        out_shape=[
            jax.ShapeDtypeStruct((_EP, 64), f32),
            jax.ShapeDtypeStruct((_EP, 128), f32),
            jax.ShapeDtypeStruct((_EP, 128), f32),
        ],
    )(ea4, nrow0, ncol0,
      wenc1, benc1, wenc2, benc2, wenc3, benc3,
      wmm1, bmm1, wmm2, bmm2, wmm3, bmm3, wmm4, bmm4)

    fo_p, fi_p = sc_scatter(fo.reshape(N_EDGES_C, 32), fi.reshape(N_EDGES_C, 32),
                            rfo, rfi, zeros_seg)

    node1 = pl.pallas_call(
        _tc_node_mlp,
        out_shape=jax.ShapeDtypeStruct((_NN_PAD // 4, 128), f32),
    )(fo_p.reshape(_NC, _NN_PAD // 4, 128), fi_p.reshape(_NC, _NN_PAD // 4, 128),
      win4, wout4, bnm4)

    nrow1, ncol1 = sc_gather(node1.reshape(_NN_PAD, 32), row2, col2)
    nrow1 = nrow1.reshape(_EP, 128)
    ncol1 = ncol1.reshape(_EP, 128)

    out = pl.pallas_call(
        _tc_step2,
        grid=(_GRID,),
        in_specs=[
            _row_spec(24), _row_spec(128), _row_spec(128), _row_spec(64),
            _full_spec(wenc1.shape), _full_spec(benc1.shape),
            _full_spec(wenc2.shape), _full_spec(benc2.shape),
            _full_spec(wenc3.shape), _full_spec(benc3.shape),
            _full_spec(wmm1h.shape), _full_spec(bmm1.shape),
            _full_spec(wmm2.shape), _full_spec(bmm2.shape),
            _full_spec(wcl1.shape), _full_spec(bcl1_4.shape),
            _full_spec(wcl2.shape), _full_spec(bcl2_4.shape),
        ],
        out_specs=[_row_spec(4)],
        out_shape=[jax.ShapeDtypeStruct((_EP, 4), f32)],
    )(ea4, nrow1, ncol1, e1,
      wenc1, benc1, wenc2, benc2, wenc3, benc3,
      wmm1h, bmm1, wmm2, bmm2, wcl1, bcl1_4, wcl2, bcl2_4)[0]

    return out.reshape(N_EDGES_C, 1)


# speedup vs baseline: 6.3978x; 1.0812x over previous
import functools

import jax
import jax.numpy as jnp
from jax import lax
from jax.experimental import pallas as pl
from jax.experimental.pallas import tpu as pltpu
from jax.experimental.pallas import tpu_sc as plsc

N_NODES_C = 10000
N_EDGES_C = 320000

_NC = 2
_NS = 16
_NW = _NC * _NS
_EW = N_EDGES_C // _NW
_G = 80
_GI = _EW // _G
_GRP = 5
_CH = _G * _GRP
_KO = _GI // _GRP
_NN_PAD = 10240
_SEG = _NN_PAD // _NS
_DUMP = 10208


@functools.lru_cache(maxsize=None)
def _sc_kernels():
    mesh = plsc.VectorSubcoreMesh(core_axis_name="c", subcore_axis_name="s")
    sc_params = pltpu.CompilerParams(use_tc_tiling_on_sc=False)

    @functools.partial(
        pl.kernel,
        mesh=mesh,
        out_type=[
            jax.ShapeDtypeStruct((N_EDGES_C, 32), jnp.float32),
            jax.ShapeDtypeStruct((N_EDGES_C, 32), jnp.float32),
        ],
        scratch_types=[
            pltpu.VMEM((_GI, _G), jnp.int32),
            pltpu.VMEM((_GI, _G), jnp.int32),
            pltpu.VMEM((2, _CH, 32), jnp.float32),
            pltpu.VMEM((2, _CH, 32), jnp.float32),
            pltpu.SemaphoreType.DMA,
            pltpu.SemaphoreType.DMA,
        ],
        compiler_params=sc_params,
    )
    def _sc_gather(node_hbm, row2_hbm, col2_hbm, nrow_hbm, ncol_hbm,
                   idxr, idxc, bufr, bufc, semr, semc):
        cid = lax.axis_index("c")
        sid = lax.axis_index("s")
        wid = sid * _NC + cid
        pltpu.sync_copy(row2_hbm.at[wid], idxr)
        pltpu.sync_copy(col2_hbm.at[wid], idxc)

        def issue(k, b):
            for g in range(_GRP):
                j = k * _GRP + g
                pltpu.async_copy(node_hbm.at[idxr.at[j]],
                                 bufr.at[b, pl.ds(g * _G, _G)], semr)
                pltpu.async_copy(node_hbm.at[idxc.at[j]],
                                 bufc.at[b, pl.ds(g * _G, _G)], semc)

        def drain(b):
            for g in range(_GRP):
                pltpu.make_async_copy(node_hbm.at[idxr.at[0]],
                                      bufr.at[b, pl.ds(g * _G, _G)], semr).wait()
                pltpu.make_async_copy(node_hbm.at[idxc.at[0]],
                                      bufc.at[b, pl.ds(g * _G, _G)], semc).wait()

        def wb(k, b):
            base_e = wid * _EW + k * _CH
            pltpu.sync_copy(bufr.at[b], nrow_hbm.at[pl.ds(base_e, _CH)])
            pltpu.sync_copy(bufc.at[b], ncol_hbm.at[pl.ds(base_e, _CH)])

        issue(0, 0)

        def outer(ko, carry):
            for b in (0, 1):
                k = 2 * ko + b
                drain(b)
                issue(k + 1, 1 - b)
                wb(k, b)
            return carry

        lax.fori_loop(0, _KO // 2, outer, 0)
        drain(0)
        wb(_KO - 1, 0)

    @functools.partial(
        pl.kernel,
        mesh=mesh,
        out_type=[
            jax.ShapeDtypeStruct((_NC, _NN_PAD, 32), jnp.float32),
            jax.ShapeDtypeStruct((_NC, _NN_PAD, 32), jnp.float32),
        ],
        scratch_types=[
            pltpu.VMEM((_GI, _G), jnp.int32),
            pltpu.VMEM((_GI, _G), jnp.int32),
            pltpu.VMEM((2, _CH, 32), jnp.float32),
            pltpu.VMEM((2, _CH, 32), jnp.float32),
            pltpu.SemaphoreType.DMA,
            pltpu.VMEM_SHARED((_NN_PAD, 32), jnp.float32),
            pltpu.VMEM_SHARED((_NN_PAD, 32), jnp.float32),
        ],
        compiler_params=sc_params,
    )
    def _sc_scatter(fo_hbm, fi_hbm, rfo_hbm, rfi_hbm, zeros_hbm,
                    outfo_hbm, outfi_hbm,
                    idxo, idxi, vfo, vfi, semld, acc_fo, acc_fi):
        cid = lax.axis_index("c")
        sid = lax.axis_index("s")
        wid = sid * _NC + cid
        seg0 = sid * _SEG
        pltpu.sync_copy(zeros_hbm, vfo.at[0])
        for acc in (acc_fo, acc_fi):
            pltpu.sync_copy(vfo.at[0], acc.at[pl.ds(seg0, _CH)])
            pltpu.sync_copy(vfo.at[0, pl.ds(0, _SEG - _CH)],
                            acc.at[pl.ds(seg0 + _CH, _SEG - _CH)])
        pltpu.sync_copy(rfo_hbm.at[wid], idxo)
        pltpu.sync_copy(rfi_hbm.at[wid], idxi)
        plsc.subcore_barrier()

        def load(k, b):
            base_e = wid * _EW + k * _CH
            pltpu.async_copy(fo_hbm.at[pl.ds(base_e, _CH)], vfo.at[b], semld)
            pltpu.async_copy(fi_hbm.at[pl.ds(base_e, _CH)], vfi.at[b], semld)

        def drain(b):
            pltpu.make_async_copy(fo_hbm.at[pl.ds(0, _CH)], vfo.at[b], semld).wait()
            pltpu.make_async_copy(fi_hbm.at[pl.ds(0, _CH)], vfi.at[b], semld).wait()

        def adds(k, b):
            for g in range(_GRP):
                j = k * _GRP + g
                pltpu.sync_copy(vfo.at[b, pl.ds(g * _G, _G)],
                                acc_fo.at[idxo.at[j]], add=True)
                pltpu.sync_copy(vfi.at[b, pl.ds(g * _G, _G)],
                                acc_fi.at[idxi.at[j]], add=True)

        load(0, 0)

        def outer(ko, carry):
            for b in (0, 1):
                k = 2 * ko + b
                drain(b)
                load(k + 1, 1 - b)
                adds(k, b)
            return carry

        lax.fori_loop(0, _KO // 2, outer, 0)
        drain(0)
        adds(_KO - 1, 0)
        plsc.subcore_barrier()
        for acc, outref in ((acc_fo, outfo_hbm), (acc_fi, outfi_hbm)):
            pltpu.sync_copy(acc.at[pl.ds(seg0, _CH)], vfo.at[0])
            pltpu.sync_copy(vfo.at[0], outref.at[cid, pl.ds(seg0, _CH)])
            pltpu.sync_copy(acc.at[pl.ds(seg0 + _CH, _SEG - _CH)],
                            vfo.at[0, pl.ds(0, _SEG - _CH)])
            pltpu.sync_copy(vfo.at[0, pl.ds(0, _SEG - _CH)],
                            outref.at[cid, pl.ds(seg0 + _CH, _SEG - _CH)])

    return _sc_gather, _sc_scatter


_BE = 6400
_BR = _BE // 4
_GRID = N_EDGES_C // _BE
_EP = N_EDGES_C // 4


def _tc_node_enc(x_ref, w1, b1, w2, b2, o_ref):
    h = jnp.maximum(jnp.dot(x_ref[...], w1[...],
                            preferred_element_type=jnp.float32) + b1[...], 0.0)
    o_ref[...] = jnp.maximum(jnp.dot(h, w2[...],
                                     preferred_element_type=jnp.float32) + b2[...], 0.0)


def _tc_step1(ea, nrow, ncol,
              wenc1, benc1, wenc2, benc2, wenc3, benc3,
              wmm1, bmm1, wmm2, bmm2, wmm3, bmm3, wmm4, bmm4,
              e1_ref, fo_ref, fi_ref):
    f32 = jnp.float32
    ie = jnp.maximum(jnp.dot(ea[...], wenc1[...], preferred_element_type=f32) + benc1[...], 0.0)
    ie = jnp.maximum(jnp.dot(ie, wenc2[...], preferred_element_type=f32) + benc2[...], 0.0)
    ie = jnp.maximum(jnp.dot(ie, wenc3[...], preferred_element_type=f32) + benc3[...], 0.0)
    x1 = jnp.concatenate([nrow[...], ncol[...], ie], axis=1)
    h = jnp.maximum(jnp.dot(x1, wmm1[...], preferred_element_type=f32) + bmm1[...], 0.0)
    e1 = jnp.maximum(jnp.dot(h, wmm2[...], preferred_element_type=f32) + bmm2[...], 0.0)
    e1_ref[...] = e1
    x3 = jnp.concatenate([ncol[...], e1], axis=1)
    tu = jnp.maximum(jnp.dot(x3, wmm3[...], preferred_element_type=f32) + bmm3[...], 0.0)
    ff = jnp.maximum(jnp.dot(tu, wmm4[...], preferred_element_type=f32) + bmm4[...], 0.0)
    fo_ref[...] = ff[:, :128]
    fi_ref[...] = ff[:, 128:]


def _tc_node_mlp(fop, fip, win4, wout4, bnm4, o_ref):
    f32 = jnp.float32
    flow_in = fip[0] + fip[1]
    flow_out = fop[0] + fop[1]
    o_ref[...] = jnp.maximum(
        jnp.dot(flow_in, win4[...], preferred_element_type=f32)
        + jnp.dot(flow_out, wout4[...], preferred_element_type=f32) + bnm4[...], 0.0)


def _tc_step2(ea, nrow, ncol, e1,
              wenc1, benc1, wenc2, benc2, wenc3, benc3,
              wmm1, bmm1, wmm2, bmm2, wcl1, bcl1, wcl2, bcl2,
              out_ref):
    f32 = jnp.float32
    ie = jnp.maximum(jnp.dot(ea[...], wenc1[...], preferred_element_type=f32) + benc1[...], 0.0)
    ie = jnp.maximum(jnp.dot(ie, wenc2[...], preferred_element_type=f32) + benc2[...], 0.0)
    ie = jnp.maximum(jnp.dot(ie, wenc3[...], preferred_element_type=f32) + benc3[...], 0.0)
    x1 = jnp.concatenate([nrow[...], ncol[...], ie, e1[...]], axis=1)
    h = jnp.maximum(jnp.dot(x1, wmm1[...], preferred_element_type=f32) + bmm1[...], 0.0)
    e2 = jnp.maximum(jnp.dot(h, wmm2[...], preferred_element_type=f32) + bmm2[...], 0.0)
    cc = jnp.maximum(jnp.dot(e2, wcl1[...], preferred_element_type=f32) + bcl1[...], 0.0)
    out_ref[...] = jnp.dot(cc, wcl2[...], preferred_element_type=f32) + bcl2[...]


def _full_spec(shape):
    return pl.BlockSpec(shape, lambda i: tuple(0 for _ in shape))


def _row_spec(cols):
    return pl.BlockSpec((_BR, cols), lambda i: (i, 0))


def _bd4(w):
    a, b = w.shape
    z = jnp.zeros((4 * a, 4 * b), jnp.float32)
    for g in range(4):
        z = z.at[g * a:(g + 1) * a, g * b:(g + 1) * b].set(w)
    return z


def kernel(x, edge_index, edge_attr, Wn1, bn1, Wn2, bn2, We1, be1, We2, be2,
           We3, be3, Wc1, bc1, Wc2, bc2, Wfi1, bfi1, Wfi2, bfi2, Wfo1, bfo1,
           Wfo2, bfo2, Wnm, bnm, Wcl1, bcl1, Wcl2, bcl2):
    f32 = jnp.float32
    row = edge_index[0].astype(jnp.int32)
    col = edge_index[1].astype(jnp.int32)
    row2 = row.reshape(_NW, _GI, _G)
    col2 = col.reshape(_NW, _GI, _G)
    dump = _DUMP + (jnp.arange(N_EDGES_C, dtype=jnp.int32) % 32)
    rfo = jnp.where(row < col, row, dump).reshape(_NW, _GI, _G)
    rfi = jnp.where(row > col, row, dump).reshape(_NW, _GI, _G)

    a1 = Wc1[0:32]
    a2 = Wc1[32:64]
    a3 = Wc1[64:80]
    a4 = Wc1[80:96]
    wenc1, benc1 = _bd4(We1), jnp.tile(be1, 4)
    wenc2, benc2 = _bd4(We2), jnp.tile(be2, 4)
    wenc3, benc3 = _bd4(We3), jnp.tile(be3, 4)
    wmm1 = jnp.concatenate([_bd4(a1), _bd4(a2), _bd4(a3 + a4)], axis=0)
    bmm1 = jnp.tile(bc1, 4)
    wmm1h = jnp.concatenate([_bd4(a1), _bd4(a2), _bd4(a3), _bd4(a4)], axis=0)
    wmm2, bmm2 = _bd4(Wc2), jnp.tile(bc2, 4)
    top = jnp.concatenate([Wfo1[:32], Wfi1[:32]], axis=1)
    bot = jnp.concatenate([Wfo1[32:], Wfi1[32:]], axis=1)
    wmm3 = jnp.concatenate([_bd4(top), _bd4(bot)], axis=0)
    bmm3 = jnp.tile(jnp.concatenate([bfo1, bfi1]), 4)
    wmm4 = jnp.zeros((448, 256), f32)
    for g in range(4):
        wmm4 = wmm4.at[112 * g:112 * g + 56, 32 * g:32 * g + 32].set(Wfo2)
        wmm4 = wmm4.at[112 * g + 56:112 * g + 112,
                       128 + 32 * g:128 + 32 * g + 32].set(Wfi2)
    bmm4 = jnp.concatenate([jnp.tile(bfo2, 4), jnp.tile(bfi2, 4)])
    win4, wout4 = _bd4(Wnm[:32]), _bd4(Wnm[32:])
    bnm4 = jnp.tile(bnm, 4)
    wcl1, bcl1_4 = _bd4(Wcl1), jnp.tile(bcl1, 4)
    wcl2, bcl2_4 = _bd4(Wcl2), jnp.tile(bcl2, 4)
    zeros_seg = jnp.zeros((_CH, 32), f32)
    ea4 = edge_attr.reshape(_EP, 24)

    node0 = pl.pallas_call(
        _tc_node_enc,
        out_shape=jax.ShapeDtypeStruct((N_NODES_C, 32), f32),
    )(x, Wn1, bn1, Wn2, bn2)

    sc_gather, sc_scatter = _sc_kernels()

    nrow0, ncol0 = sc_gather(node0, row2, col2)
    nrow0 = nrow0.reshape(_EP, 128)
    ncol0 = ncol0.reshape(_EP, 128)

    e1, fo, fi = pl.pallas_call(
        _tc_step1,
        grid=(_GRID,),
        in_specs=[
            _row_spec(24), _row_spec(128), _row_spec(128),
            _full_spec(wenc1.shape), _full_spec(benc1.shape),
            _full_spec(wenc2.shape), _full_spec(benc2.shape),
            _full_spec(wenc3.shape), _full_spec(benc3.shape),
            _full_spec(wmm1.shape), _full_spec(bmm1.shape),
            _full_spec(wmm2.shape), _full_spec(bmm2.shape),
            _full_spec(wmm3.shape), _full_spec(bmm3.shape),
            _full_spec(wmm4.shape), _full_spec(bmm4.shape),
        ],
        out_specs=[_row_spec(64), _row_spec(128), _row_spec(128)],
        out_shape=[
            jax.ShapeDtypeStruct((_EP, 64), f32),
            jax.ShapeDtypeStruct((_EP, 128), f32),
            jax.ShapeDtypeStruct((_EP, 128), f32),
        ],
    )(ea4, nrow0, ncol0,
      wenc1, benc1, wenc2, benc2, wenc3, benc3,
      wmm1, bmm1, wmm2, bmm2, wmm3, bmm3, wmm4, bmm4)

    fo_p, fi_p = sc_scatter(fo.reshape(N_EDGES_C, 32), fi.reshape(N_EDGES_C, 32),
                            rfo, rfi, zeros_seg)

    node1 = pl.pallas_call(
        _tc_node_mlp,
        out_shape=jax.ShapeDtypeStruct((_NN_PAD // 4, 128), f32),
    )(fo_p.reshape(_NC, _NN_PAD // 4, 128), fi_p.reshape(_NC, _NN_PAD // 4, 128),
      win4, wout4, bnm4)

    nrow1, ncol1 = sc_gather(node1.reshape(_NN_PAD, 32), row2, col2)
    nrow1 = nrow1.reshape(_EP, 128)
    ncol1 = ncol1.reshape(_EP, 128)

    out = pl.pallas_call(
        _tc_step2,
        grid=(_GRID,),
        in_specs=[
            _row_spec(24), _row_spec(128), _row_spec(128), _row_spec(64),
            _full_spec(wenc1.shape), _full_spec(benc1.shape),
            _full_spec(wenc2.shape), _full_spec(benc2.shape),
            _full_spec(wenc3.shape), _full_spec(benc3.shape),
            _full_spec(wmm1h.shape), _full_spec(bmm1.shape),
            _full_spec(wmm2.shape), _full_spec(bmm2.shape),
            _full_spec(wcl1.shape), _full_spec(bcl1_4.shape),
            _full_spec(wcl2.shape), _full_spec(bcl2_4.shape),
        ],
        out_specs=[_row_spec(4)],
        out_shape=[jax.ShapeDtypeStruct((_EP, 4), f32)],
    )(ea4, nrow1, ncol1, e1,
      wenc1, benc1, wenc2, benc2, wenc3, benc3,
      wmm1h, bmm1, wmm2, bmm2, wcl1, bcl1_4, wcl2, bcl2_4)[0]

    return out.reshape(N_EDGES_C, 1)


# speedup vs baseline: 6.6859x; 1.0450x over previous
import functools

import jax
import jax.numpy as jnp
from jax import lax
from jax.experimental import pallas as pl
from jax.experimental.pallas import tpu as pltpu
from jax.experimental.pallas import tpu_sc as plsc

N_NODES_C = 10000
N_EDGES_C = 320000

_NC = 2
_NS = 16
_NW = _NC * _NS
_EW = N_EDGES_C // _NW
_G = 80
_GI = _EW // _G
_GRP = 5
_CH = _G * _GRP
_KO = _GI // _GRP
_NN_PAD = 10240
_SEG = _NN_PAD // _NS
_DUMP = 10208


@functools.lru_cache(maxsize=None)
def _sc_kernels():
    mesh = plsc.VectorSubcoreMesh(core_axis_name="c", subcore_axis_name="s")
    sc_params = pltpu.CompilerParams(use_tc_tiling_on_sc=False)

    @functools.partial(
        pl.kernel,
        mesh=mesh,
        out_type=[
            jax.ShapeDtypeStruct((N_EDGES_C, 32), jnp.float32),
            jax.ShapeDtypeStruct((N_EDGES_C, 32), jnp.float32),
        ],
        scratch_types=[
            pltpu.VMEM((_GI, _G), jnp.int32),
            pltpu.VMEM((_GI, _G), jnp.int32),
            pltpu.VMEM((2, _CH, 32), jnp.float32),
            pltpu.VMEM((2, _CH, 32), jnp.float32),
            pltpu.SemaphoreType.DMA,
            pltpu.SemaphoreType.DMA,
        ],
        compiler_params=sc_params,
    )
    def _sc_gather(node_hbm, row2_hbm, col2_hbm, nrow_hbm, ncol_hbm,
                   idxr, idxc, bufr, bufc, semr, semc):
        cid = lax.axis_index("c")
        sid = lax.axis_index("s")
        wid = sid * _NC + cid
        pltpu.sync_copy(row2_hbm.at[wid], idxr)
        pltpu.sync_copy(col2_hbm.at[wid], idxc)

        def issue(k, b):
            for g in range(_GRP):
                j = k * _GRP + g
                pltpu.async_copy(node_hbm.at[idxr.at[j]],
                                 bufr.at[b, pl.ds(g * _G, _G)], semr)
                pltpu.async_copy(node_hbm.at[idxc.at[j]],
                                 bufc.at[b, pl.ds(g * _G, _G)], semc)

        def drain(b):
            for g in range(_GRP):
                pltpu.make_async_copy(node_hbm.at[idxr.at[0]],
                                      bufr.at[b, pl.ds(g * _G, _G)], semr).wait()
                pltpu.make_async_copy(node_hbm.at[idxc.at[0]],
                                      bufc.at[b, pl.ds(g * _G, _G)], semc).wait()

        def wb(k, b):
            base_e = wid * _EW + k * _CH
            pltpu.sync_copy(bufr.at[b], nrow_hbm.at[pl.ds(base_e, _CH)])
            pltpu.sync_copy(bufc.at[b], ncol_hbm.at[pl.ds(base_e, _CH)])

        issue(0, 0)

        def outer(ko, carry):
            for b in (0, 1):
                k = 2 * ko + b
                drain(b)
                issue(k + 1, 1 - b)
                wb(k, b)
            return carry

        lax.fori_loop(0, _KO // 2, outer, 0)
        drain(0)
        wb(_KO - 1, 0)

    @functools.partial(
        pl.kernel,
        mesh=mesh,
        out_type=[
            jax.ShapeDtypeStruct((_NC, _NN_PAD, 32), jnp.float32),
            jax.ShapeDtypeStruct((_NC, _NN_PAD, 32), jnp.float32),
        ],
        scratch_types=[
            pltpu.VMEM((_GI, _G), jnp.int32),
            pltpu.VMEM((_GI, _G), jnp.int32),
            pltpu.VMEM((2, _CH, 32), jnp.float32),
            pltpu.VMEM((2, _CH, 32), jnp.float32),
            pltpu.SemaphoreType.DMA,
            pltpu.VMEM_SHARED((_NN_PAD, 32), jnp.float32),
            pltpu.VMEM_SHARED((_NN_PAD, 32), jnp.float32),
        ],
        compiler_params=sc_params,
    )
    def _sc_scatter(fo_hbm, fi_hbm, rfo_hbm, rfi_hbm, zeros_hbm,
                    outfo_hbm, outfi_hbm,
                    idxo, idxi, vfo, vfi, semld, acc_fo, acc_fi):
        cid = lax.axis_index("c")
        sid = lax.axis_index("s")
        wid = sid * _NC + cid
        seg0 = sid * _SEG
        pltpu.sync_copy(zeros_hbm, vfo.at[0])
        for acc in (acc_fo, acc_fi):
            pltpu.sync_copy(vfo.at[0], acc.at[pl.ds(seg0, _CH)])
            pltpu.sync_copy(vfo.at[0, pl.ds(0, _SEG - _CH)],
                            acc.at[pl.ds(seg0 + _CH, _SEG - _CH)])
        pltpu.sync_copy(rfo_hbm.at[wid], idxo)
        pltpu.sync_copy(rfi_hbm.at[wid], idxi)
        plsc.subcore_barrier()

        def load(k, b):
            base_e = wid * _EW + k * _CH
            pltpu.async_copy(fo_hbm.at[pl.ds(base_e, _CH)], vfo.at[b], semld)
            pltpu.async_copy(fi_hbm.at[pl.ds(base_e, _CH)], vfi.at[b], semld)

        def drain(b):
            pltpu.make_async_copy(fo_hbm.at[pl.ds(0, _CH)], vfo.at[b], semld).wait()
            pltpu.make_async_copy(fi_hbm.at[pl.ds(0, _CH)], vfi.at[b], semld).wait()

        def adds(k, b):
            for g in range(_GRP):
                j = k * _GRP + g
                pltpu.sync_copy(vfo.at[b, pl.ds(g * _G, _G)],
                                acc_fo.at[idxo.at[j]], add=True)
                pltpu.sync_copy(vfi.at[b, pl.ds(g * _G, _G)],
                                acc_fi.at[idxi.at[j]], add=True)

        load(0, 0)

        def outer(ko, carry):
            for b in (0, 1):
                k = 2 * ko + b
                drain(b)
                load(k + 1, 1 - b)
                adds(k, b)
            return carry

        lax.fori_loop(0, _KO // 2, outer, 0)
        drain(0)
        adds(_KO - 1, 0)
        plsc.subcore_barrier()
        for acc, outref in ((acc_fo, outfo_hbm), (acc_fi, outfi_hbm)):
            pltpu.sync_copy(acc.at[pl.ds(seg0, _CH)], vfo.at[0])
            pltpu.sync_copy(vfo.at[0], outref.at[cid, pl.ds(seg0, _CH)])
            pltpu.sync_copy(acc.at[pl.ds(seg0 + _CH, _SEG - _CH)],
                            vfo.at[0, pl.ds(0, _SEG - _CH)])
            pltpu.sync_copy(vfo.at[0, pl.ds(0, _SEG - _CH)],
                            outref.at[cid, pl.ds(seg0 + _CH, _SEG - _CH)])

    return _sc_gather, _sc_scatter


_BE = 12800
_BR = _BE // 4
_GRID = N_EDGES_C // _BE
_EP = N_EDGES_C // 4


def _tc_node_enc(x_ref, w1, b1, w2, b2, o_ref):
    h = jnp.maximum(jnp.dot(x_ref[...], w1[...],
                            preferred_element_type=jnp.float32) + b1[...], 0.0)
    o_ref[...] = jnp.maximum(jnp.dot(h, w2[...],
                                     preferred_element_type=jnp.float32) + b2[...], 0.0)


def _tc_step1(ea, nrow, ncol,
              wenc1, benc1, wenc2, benc2, wenc3, benc3,
              wmm1, bmm1, wmm2, bmm2, wmm3, bmm3, wmm4, bmm4,
              e1_ref, fo_ref, fi_ref):
    f32 = jnp.float32
    ie = jnp.maximum(jnp.dot(ea[...], wenc1[...], preferred_element_type=f32) + benc1[...], 0.0)
    ie = jnp.maximum(jnp.dot(ie, wenc2[...], preferred_element_type=f32) + benc2[...], 0.0)
    ie = jnp.maximum(jnp.dot(ie, wenc3[...], preferred_element_type=f32) + benc3[...], 0.0)
    x1 = jnp.concatenate([nrow[...], ncol[...], ie], axis=1)
    h = jnp.maximum(jnp.dot(x1, wmm1[...], preferred_element_type=f32) + bmm1[...], 0.0)
    e1 = jnp.maximum(jnp.dot(h, wmm2[...], preferred_element_type=f32) + bmm2[...], 0.0)
    e1_ref[...] = e1
    x3 = jnp.concatenate([ncol[...], e1], axis=1)
    tu = jnp.maximum(jnp.dot(x3, wmm3[...], preferred_element_type=f32) + bmm3[...], 0.0)
    ff = jnp.maximum(jnp.dot(tu, wmm4[...], preferred_element_type=f32) + bmm4[...], 0.0)
    fo_ref[...] = ff[:, :128]
    fi_ref[...] = ff[:, 128:]


def _tc_node_mlp(fop, fip, win4, wout4, bnm4, o_ref):
    f32 = jnp.float32
    flow_in = fip[0] + fip[1]
    flow_out = fop[0] + fop[1]
    o_ref[...] = jnp.maximum(
        jnp.dot(flow_in, win4[...], preferred_element_type=f32)
        + jnp.dot(flow_out, wout4[...], preferred_element_type=f32) + bnm4[...], 0.0)


def _tc_step2(ea, nrow, ncol, e1,
              wenc1, benc1, wenc2, benc2, wenc3, benc3,
              wmm1, bmm1, wmm2, bmm2, wcl1, bcl1, wcl2, bcl2,
              out_ref):
    f32 = jnp.float32
    ie = jnp.maximum(jnp.dot(ea[...], wenc1[...], preferred_element_type=f32) + benc1[...], 0.0)
    ie = jnp.maximum(jnp.dot(ie, wenc2[...], preferred_element_type=f32) + benc2[...], 0.0)
    ie = jnp.maximum(jnp.dot(ie, wenc3[...], preferred_element_type=f32) + benc3[...], 0.0)
    x1 = jnp.concatenate([nrow[...], ncol[...], ie, e1[...]], axis=1)
    h = jnp.maximum(jnp.dot(x1, wmm1[...], preferred_element_type=f32) + bmm1[...], 0.0)
    e2 = jnp.maximum(jnp.dot(h, wmm2[...], preferred_element_type=f32) + bmm2[...], 0.0)
    cc = jnp.maximum(jnp.dot(e2, wcl1[...], preferred_element_type=f32) + bcl1[...], 0.0)
    out_ref[...] = jnp.dot(cc, wcl2[...], preferred_element_type=f32) + bcl2[...]


def _full_spec(shape):
    return pl.BlockSpec(shape, lambda i: tuple(0 for _ in shape))


def _row_spec(cols):
    return pl.BlockSpec((_BR, cols), lambda i: (i, 0))


def _bd4(w):
    a, b = w.shape
    z = jnp.zeros((4 * a, 4 * b), jnp.float32)
    for g in range(4):
        z = z.at[g * a:(g + 1) * a, g * b:(g + 1) * b].set(w)
    return z


def kernel(x, edge_index, edge_attr, Wn1, bn1, Wn2, bn2, We1, be1, We2, be2,
           We3, be3, Wc1, bc1, Wc2, bc2, Wfi1, bfi1, Wfi2, bfi2, Wfo1, bfo1,
           Wfo2, bfo2, Wnm, bnm, Wcl1, bcl1, Wcl2, bcl2):
    f32 = jnp.float32
    row = edge_index[0].astype(jnp.int32)
    col = edge_index[1].astype(jnp.int32)
    row2 = row.reshape(_NW, _GI, _G)
    col2 = col.reshape(_NW, _GI, _G)
    dump = _DUMP + (jnp.arange(N_EDGES_C, dtype=jnp.int32) % 32)
    rfo = jnp.where(row < col, row, dump).reshape(_NW, _GI, _G)
    rfi = jnp.where(row > col, row, dump).reshape(_NW, _GI, _G)

    a1 = Wc1[0:32]
    a2 = Wc1[32:64]
    a3 = Wc1[64:80]
    a4 = Wc1[80:96]
    wenc1, benc1 = _bd4(We1), jnp.tile(be1, 4)
    wenc2, benc2 = _bd4(We2), jnp.tile(be2, 4)
    wenc3, benc3 = _bd4(We3), jnp.tile(be3, 4)
    wmm1 = jnp.concatenate([_bd4(a1), _bd4(a2), _bd4(a3 + a4)], axis=0)
    bmm1 = jnp.tile(bc1, 4)
    wmm1h = jnp.concatenate([_bd4(a1), _bd4(a2), _bd4(a3), _bd4(a4)], axis=0)
    wmm2, bmm2 = _bd4(Wc2), jnp.tile(bc2, 4)
    top = jnp.concatenate([Wfo1[:32], Wfi1[:32]], axis=1)
    bot = jnp.concatenate([Wfo1[32:], Wfi1[32:]], axis=1)
    wmm3 = jnp.concatenate([_bd4(top), _bd4(bot)], axis=0)
    bmm3 = jnp.tile(jnp.concatenate([bfo1, bfi1]), 4)
    wmm4 = jnp.zeros((448, 256), f32)
    for g in range(4):
        wmm4 = wmm4.at[112 * g:112 * g + 56, 32 * g:32 * g + 32].set(Wfo2)
        wmm4 = wmm4.at[112 * g + 56:112 * g + 112,
                       128 + 32 * g:128 + 32 * g + 32].set(Wfi2)
    bmm4 = jnp.concatenate([jnp.tile(bfo2, 4), jnp.tile(bfi2, 4)])
    win4, wout4 = _bd4(Wnm[:32]), _bd4(Wnm[32:])
    bnm4 = jnp.tile(bnm, 4)
    wcl1, bcl1_4 = _bd4(Wcl1), jnp.tile(bcl1, 4)
    wcl2, bcl2_4 = _bd4(Wcl2), jnp.tile(bcl2, 4)
    zeros_seg = jnp.zeros((_CH, 32), f32)
    ea4 = edge_attr.reshape(_EP, 24)

    node0 = pl.pallas_call(
        _tc_node_enc,
        out_shape=jax.ShapeDtypeStruct((N_NODES_C, 32), f32),
    )(x, Wn1, bn1, Wn2, bn2)

    sc_gather, sc_scatter = _sc_kernels()

    nrow0, ncol0 = sc_gather(node0, row2, col2)
    nrow0 = nrow0.reshape(_EP, 128)
    ncol0 = ncol0.reshape(_EP, 128)

    e1, fo, fi = pl.pallas_call(
        _tc_step1,
        grid=(_GRID,),
        in_specs=[
            _row_spec(24), _row_spec(128), _row_spec(128),
            _full_spec(wenc1.shape), _full_spec(benc1.shape),
            _full_spec(wenc2.shape), _full_spec(benc2.shape),
            _full_spec(wenc3.shape), _full_spec(benc3.shape),
            _full_spec(wmm1.shape), _full_spec(bmm1.shape),
            _full_spec(wmm2.shape), _full_spec(bmm2.shape),
            _full_spec(wmm3.shape), _full_spec(bmm3.shape),
            _full_spec(wmm4.shape), _full_spec(bmm4.shape),
        ],
        out_specs=[_row_spec(64), _row_spec(128), _row_spec(128)],
        out_shape=[
            jax.ShapeDtypeStruct((_EP, 64), f32),
            jax.ShapeDtypeStruct((_EP, 128), f32),
            jax.ShapeDtypeStruct((_EP, 128), f32),
        ],
    )(ea4, nrow0, ncol0,
      wenc1, benc1, wenc2, benc2, wenc3, benc3,
      wmm1, bmm1, wmm2, bmm2, wmm3, bmm3, wmm4, bmm4)

    fo_p, fi_p = sc_scatter(fo.reshape(N_EDGES_C, 32), fi.reshape(N_EDGES_C, 32),
                            rfo, rfi, zeros_seg)

    node1 = pl.pallas_call(
        _tc_node_mlp,
        out_shape=jax.ShapeDtypeStruct((_NN_PAD // 4, 128), f32),
    )(fo_p.reshape(_NC, _NN_PAD // 4, 128), fi_p.reshape(_NC, _NN_PAD // 4, 128),
      win4, wout4, bnm4)

    nrow1, ncol1 = sc_gather(node1.reshape(_NN_PAD, 32), row2, col2)
    nrow1 = nrow1.reshape(_EP, 128)
    ncol1 = ncol1.reshape(_EP, 128)

    out = pl.pallas_call(
        _tc_step2,
        grid=(_GRID,),
        in_specs=[
            _row_spec(24), _row_spec(128), _row_spec(128), _row_spec(64),
            _full_spec(wenc1.shape), _full_spec(benc1.shape),
            _full_spec(wenc2.shape), _full_spec(benc2.shape),
            _full_spec(wenc3.shape), _full_spec(benc3.shape),
            _full_spec(wmm1h.shape), _full_spec(bmm1.shape),
            _full_spec(wmm2.shape), _full_spec(bmm2.shape),
            _full_spec(wcl1.shape), _full_spec(bcl1_4.shape),
            _full_spec(wcl2.shape), _full_spec(bcl2_4.shape),
        ],
        out_specs=[_row_spec(4)],
        out_shape=[jax.ShapeDtypeStruct((_EP, 4), f32)],
    )(ea4, nrow1, ncol1, e1,
      wenc1, benc1, wenc2, benc2, wenc3, benc3,
      wmm1h, bmm1, wmm2, bmm2, wcl1, bcl1_4, wcl2, bcl2_4)[0]

    return out.reshape(N_EDGES_C, 1)
